# baseline (speedup 1.0000x reference)
import functools

import jax
import jax.numpy as jnp
from jax import lax
from jax.experimental import pallas as pl
from jax.experimental.pallas import tpu as pltpu
from jax.experimental.pallas import tpu_sc as plsc

NC = 2
NS = 16
NW = NC * NS

D = 64
TW = 128
GROUP = 8
NBUF = 2
L = 16


def _tile_body(n_per_w, hist, idx_hbm, table_hbm, out_hbm, idx_v, tab_v, obuf, ssem):
    wid = lax.axis_index("s") * NC + lax.axis_index("c")
    e0 = wid * n_per_w
    ngroups = n_per_w // GROUP
    starts = []
    r = 0
    while r + L <= hist:
        starts.append(r)
        r += L
    if starts[-1] + L < hist:
        starts.append(hist - L)

    pltpu.sync_copy(table_hbm, tab_v)

    def _expand(j, b):
        for r0 in starts:
            iv = idx_v[j, pl.ds(r0, L)]
            for l in range(L):
                i = iv[l]
                for c in range(D // L):
                    obuf[b, r0 + l, pl.ds(c * L, L)] = tab_v[i, pl.ds(c * L, L)]

    def group_step(g, carry):
        ge = e0 + g * GROUP
        pltpu.sync_copy(idx_hbm.at[pl.ds(ge, GROUP)], idx_v)

        def pair_step(p, carry2):
            for b in range(NBUF):
                j = p * NBUF + b

                def _wait_prev_store():
                    pltpu.make_async_copy(
                        obuf.at[b], out_hbm.at[0], ssem.at[b]
                    ).wait()

                pl.when((g > 0) | (p > 0))(_wait_prev_store)
                _expand(j, b)
                pltpu.async_copy(obuf.at[b], out_hbm.at[ge + j], ssem.at[b])
            return carry2

        lax.fori_loop(0, GROUP // NBUF, pair_step, 0)
        return carry

    lax.fori_loop(0, ngroups, group_step, 0)
    for b in range(NBUF):
        pltpu.make_async_copy(obuf.at[b], out_hbm.at[0], ssem.at[b]).wait()


def kernel(tc, embedding):
    bsz, hist = tc.shape
    assert bsz % (NW * GROUP) == 0
    n_per_w = bsz // NW

    idx = tc.astype(jnp.int32)
    table = jnp.pad(embedding, ((0, 0), (0, TW - embedding.shape[1])))
    mesh = plsc.VectorSubcoreMesh(
        core_axis_name="c", subcore_axis_name="s", num_cores=NC, num_subcores=NS
    )
    run = pl.kernel(
        functools.partial(_tile_body, n_per_w, hist),
        out_type=jax.ShapeDtypeStruct((bsz, hist, D), jnp.float32),
        mesh=mesh,
        scratch_types=[
            pltpu.VMEM((GROUP, hist), jnp.int32),
            pltpu.VMEM(table.shape, jnp.float32),
            pltpu.VMEM((NBUF, hist, D), jnp.float32),
            pltpu.SemaphoreType.DMA((NBUF,)),
        ],
    )
    return run(idx, table)


# speedup vs baseline: 1.3687x; 1.3687x over previous
import functools

import jax
import jax.numpy as jnp
from jax import lax
from jax.experimental import pallas as pl
from jax.experimental.pallas import tpu as pltpu
from jax.experimental.pallas import tpu_sc as plsc

NC = 2
NS = 16
NW = NC * NS

D = 64
BW = 512
SKEW = BW + 1
HG = 8
NBUF = 2
L = 16


def _tile_body(hist, idx_hbm, table_hbm, out_hbm, idx_v, tab_v, obuf, ssem):
    wid = lax.axis_index("s") * NC + lax.axis_index("c")
    b0 = wid * BW
    lane = lax.iota(jnp.int32, L)

    pltpu.sync_copy(table_hbm, tab_v)

    def _expand(hj, par):
        def chunk_step(k, carry):
            iv = idx_v[hj, pl.ds(k * L, L)]
            for l in range(L):
                i = iv[l]
                bl = jnp.full((L,), k * L + l, jnp.int32)
                for c in range(D // L):
                    vals = tab_v[i, pl.ds(c * L, L)]
                    plsc.store_scatter(obuf.at[par], [c * L + lane, bl], vals)
            return carry

        lax.fori_loop(0, BW // L, chunk_step, 0)

    def group_step(g, carry):
        h0 = g * HG
        pltpu.sync_copy(idx_hbm.at[pl.ds(h0, HG), pl.ds(b0, BW)], idx_v)

        def pair_step(hp, carry2):
            for par in range(NBUF):
                hj = hp * NBUF + par
                h = h0 + hj

                def _wait_prev_store():
                    pltpu.make_async_copy(
                        obuf.at[par, :, pl.ds(0, BW)],
                        out_hbm.at[0, :, pl.ds(b0, BW)],
                        ssem.at[par],
                    ).wait()

                pl.when((g > 0) | (hp > 0))(_wait_prev_store)
                _expand(hj, par)
                pltpu.async_copy(
                    obuf.at[par, :, pl.ds(0, BW)],
                    out_hbm.at[h, :, pl.ds(b0, BW)],
                    ssem.at[par],
                )
            return carry2

        lax.fori_loop(0, HG // NBUF, pair_step, 0)
        return carry

    lax.fori_loop(0, hist // HG, group_step, 0)
    for par in range(NBUF):
        pltpu.make_async_copy(
            obuf.at[par, :, pl.ds(0, BW)],
            out_hbm.at[0, :, pl.ds(b0, BW)],
            ssem.at[par],
        ).wait()


def kernel(tc, embedding):
    bsz, hist = tc.shape
    assert bsz % NW == 0 and bsz // NW == BW
    assert hist % HG == 0

    idx_t = jnp.transpose(tc).astype(jnp.int32)
    mesh = plsc.VectorSubcoreMesh(
        core_axis_name="c", subcore_axis_name="s", num_cores=NC, num_subcores=NS
    )
    run = pl.kernel(
        functools.partial(_tile_body, hist),
        out_type=jax.ShapeDtypeStruct((hist, D, bsz), jnp.float32),
        mesh=mesh,
        scratch_types=[
            pltpu.VMEM((HG, BW), jnp.int32),
            pltpu.VMEM(embedding.shape, jnp.float32),
            pltpu.VMEM((NBUF, D, SKEW), jnp.float32),
            pltpu.SemaphoreType.DMA((NBUF,)),
        ],
        compiler_params=pltpu.CompilerParams(
            use_tc_tiling_on_sc=False, needs_layout_passes=False
        ),
    )
    out_t = run(idx_t, embedding)
    return jnp.transpose(out_t, (2, 0, 1))


# speedup vs baseline: 1.8768x; 1.3712x over previous
import functools

import jax
import jax.numpy as jnp
from jax import lax
from jax.experimental import pallas as pl
from jax.experimental.pallas import tpu as pltpu
from jax.experimental.pallas import tpu_sc as plsc

NC = 2
NS = 16
NW = NC * NS

D = 64
BW = 512
SW = 256
TSK = D + 1
HG = 8
NBUF = 2
L = 16


def _tile_body(hist, nrows, idx_hbm, table_hbm, out_hbm,
               idx_v, tab_v, tab_skew, obuf, ssem):
    wid = lax.axis_index("s") * NC + lax.axis_index("c")
    b0 = wid * BW

    pltpu.sync_copy(table_hbm, tab_v)

    def repack_step(i, carry):
        for c0 in range(0, D, L):
            tab_skew[pl.ds(i * TSK + c0, L)] = tab_v[i, pl.ds(c0, L)]
        return carry

    lax.fori_loop(0, nrows, repack_step, 0)

    def _expand(hj, off, par):
        def chunk_step(k, carry):
            iv = idx_v[hj, pl.ds(off + k * L, L)]
            ivm = iv * TSK
            for c in range(D):
                vals = plsc.load_gather(tab_skew, [ivm + c])
                obuf[par, c, pl.ds(k * L, L)] = vals
            return carry

        lax.fori_loop(0, SW // L, chunk_step, 0)

    def group_step(g, carry):
        h0 = g * HG
        pltpu.sync_copy(idx_hbm.at[pl.ds(h0, HG), pl.ds(b0, BW)], idx_v)

        def pair_step(hp, carry2):
            for par in range(NBUF):
                s = hp * NBUF + par
                hj = s // 2
                off = (s % 2) * SW
                h = h0 + hj

                def _wait_prev_store():
                    pltpu.make_async_copy(
                        obuf.at[par],
                        out_hbm.at[0, :, pl.ds(b0, SW)],
                        ssem.at[par],
                    ).wait()

                pl.when((g > 0) | (hp > 0))(_wait_prev_store)
                _expand(hj, off, par)
                pltpu.async_copy(
                    obuf.at[par],
                    out_hbm.at[h, :, pl.ds(b0 + off, SW)],
                    ssem.at[par],
                )
            return carry2

        lax.fori_loop(0, HG * 2 // NBUF, pair_step, 0)
        return carry

    lax.fori_loop(0, hist // HG, group_step, 0)
    for par in range(NBUF):
        pltpu.make_async_copy(
            obuf.at[par], out_hbm.at[0, :, pl.ds(b0, SW)], ssem.at[par]
        ).wait()


def kernel(tc, embedding):
    bsz, hist = tc.shape
    nrows = embedding.shape[0]
    assert bsz % NW == 0 and bsz // NW == BW
    assert hist % HG == 0

    idx_t = jnp.transpose(tc).astype(jnp.int32)
    mesh = plsc.VectorSubcoreMesh(
        core_axis_name="c", subcore_axis_name="s", num_cores=NC, num_subcores=NS
    )
    run = pl.kernel(
        functools.partial(_tile_body, hist, nrows),
        out_type=jax.ShapeDtypeStruct((hist, D, bsz), jnp.float32),
        mesh=mesh,
        scratch_types=[
            pltpu.VMEM((HG, BW), jnp.int32),
            pltpu.VMEM(embedding.shape, jnp.float32),
            pltpu.VMEM((nrows * TSK,), jnp.float32),
            pltpu.VMEM((NBUF, D, SW), jnp.float32),
            pltpu.SemaphoreType.DMA((NBUF,)),
        ],
        compiler_params=pltpu.CompilerParams(needs_layout_passes=False),
    )
    out_t = run(idx_t, embedding)
    return jnp.transpose(out_t, (2, 0, 1))


# speedup vs baseline: 6.1832x; 3.2945x over previous
import functools

import jax
import jax.numpy as jnp
from jax import lax
from jax.experimental import pallas as pl
from jax.experimental.pallas import tpu as pltpu
from jax.experimental.pallas import tpu_sc as plsc

NC = 2
NS = 16
NW = NC * NS

D = 64
BW = 512
SW = 256
TSK = D + 1
HG = 8
NBUF = 2
GB = 8
L = 16


def _tile_body(hist, nrows, idx_hbm, table_hbm, out_hbm,
               idx_v, tab_v, tab_skew, obuf, ssem):
    wid = lax.axis_index("s") * NC + lax.axis_index("c")
    b0 = wid * BW

    pltpu.sync_copy(table_hbm, tab_v)

    def repack_step(i, carry):
        for c0 in range(0, D, L):
            tab_skew[pl.ds(i * TSK + c0, L)] = tab_v[i, pl.ds(c0, L)]
        return carry

    lax.fori_loop(0, nrows, repack_step, 0)

    def _expand(hj, off, par):
        def chunk_step(k, carry):
            iv = idx_v[hj, pl.ds(off + k * L, L)]
            ivm = iv * TSK
            for c0 in range(0, D, GB):
                vals = [
                    plsc.load_gather(tab_skew, [ivm + (c0 + t)])
                    for t in range(GB)
                ]
                for t in range(GB):
                    obuf[par, c0 + t, pl.ds(k * L, L)] = vals[t]
            return carry

        lax.fori_loop(0, SW // L, chunk_step, 0)

    def group_step(g, carry):
        h0 = g * HG
        pltpu.sync_copy(idx_hbm.at[pl.ds(h0, HG), pl.ds(b0, BW)], idx_v)

        def pair_step(hp, carry2):
            for par in range(NBUF):
                s = hp * NBUF + par
                hj = s // 2
                off = (s % 2) * SW
                h = h0 + hj

                def _wait_prev_store():
                    pltpu.make_async_copy(
                        obuf.at[par],
                        out_hbm.at[0, :, pl.ds(b0, SW)],
                        ssem.at[par],
                    ).wait()

                pl.when((g > 0) | (hp > 0))(_wait_prev_store)
                _expand(hj, off, par)
                pltpu.async_copy(
                    obuf.at[par],
                    out_hbm.at[h, :, pl.ds(b0 + off, SW)],
                    ssem.at[par],
                )
            return carry2

        lax.fori_loop(0, HG * 2 // NBUF, pair_step, 0)
        return carry

    lax.fori_loop(0, hist // HG, group_step, 0)
    for par in range(NBUF):
        pltpu.make_async_copy(
            obuf.at[par], out_hbm.at[0, :, pl.ds(b0, SW)], ssem.at[par]
        ).wait()


def kernel(tc, embedding):
    bsz, hist = tc.shape
    nrows = embedding.shape[0]
    assert bsz % NW == 0 and bsz // NW == BW
    assert hist % HG == 0

    idx_t = jnp.transpose(tc).astype(jnp.int32)
    mesh = plsc.VectorSubcoreMesh(
        core_axis_name="c", subcore_axis_name="s", num_cores=NC, num_subcores=NS
    )
    run = pl.kernel(
        functools.partial(_tile_body, hist, nrows),
        out_type=jax.ShapeDtypeStruct((hist, D, bsz), jnp.float32),
        mesh=mesh,
        scratch_types=[
            pltpu.VMEM((HG, BW), jnp.int32),
            pltpu.VMEM(embedding.shape, jnp.float32),
            pltpu.VMEM((nrows * TSK,), jnp.float32),
            pltpu.VMEM((NBUF, D, SW), jnp.float32),
            pltpu.SemaphoreType.DMA((NBUF,)),
        ],
        compiler_params=pltpu.CompilerParams(needs_layout_passes=False),
    )
    out_t = run(idx_t, embedding)
    return jnp.transpose(out_t, (2, 0, 1))


# speedup vs baseline: 6.2370x; 1.0087x over previous
import functools

import jax
import jax.numpy as jnp
from jax import lax
from jax.experimental import pallas as pl
from jax.experimental.pallas import tpu as pltpu
from jax.experimental.pallas import tpu_sc as plsc

NC = 2
NS = 16
NW = NC * NS

D = 64
BW = 512
SW = 256
TSK = D + 1
HG = 4
NBUF = 2
GB = 8
L = 16


def _tile_body(hist, nrows, idx_hbm, table_hbm, out_hbm,
               idx_v, tab_v, tab_skew, obuf, ssem, isem):
    wid = lax.axis_index("s") * NC + lax.axis_index("c")
    b0 = wid * BW
    ngroups = hist // HG

    pltpu.sync_copy(table_hbm, tab_v)

    def repack_step(i, carry):
        for c0 in range(0, D, L):
            tab_skew[pl.ds(i * TSK + c0, L)] = tab_v[i, pl.ds(c0, L)]
        return carry

    lax.fori_loop(0, nrows, repack_step, 0)

    def _expand(ib, hj, off, par):
        def chunk_step(k, carry):
            iv = idx_v[ib, hj, pl.ds(off + k * L, L)]
            ivm = iv * TSK
            for c0 in range(0, D, GB):
                vals = [
                    plsc.load_gather(tab_skew, [ivm + (c0 + t)])
                    for t in range(GB)
                ]
                for t in range(GB):
                    obuf[par, c0 + t, pl.ds(k * L, L)] = vals[t]
            return carry

        lax.fori_loop(0, SW // L, chunk_step, 0, unroll=2)

    for ib in range(2):
        pltpu.async_copy(
            idx_hbm.at[pl.ds(ib * HG, HG), pl.ds(b0, BW)],
            idx_v.at[ib], isem.at[ib],
        )

    def gp_step(gp, carry):
        for ib in range(2):
            g = gp * 2 + ib
            h0 = g * HG
            pltpu.make_async_copy(
                idx_hbm.at[pl.ds(0, HG), pl.ds(b0, BW)],
                idx_v.at[ib], isem.at[ib],
            ).wait()

            def pair_step(hp, carry2):
                for par in range(NBUF):
                    s = hp * NBUF + par
                    hj = s // 2
                    off = (s % 2) * SW
                    h = h0 + hj

                    def _wait_prev_store():
                        pltpu.make_async_copy(
                            obuf.at[par],
                            out_hbm.at[0, :, pl.ds(b0, SW)],
                            ssem.at[par],
                        ).wait()

                    pl.when((g > 0) | (hp > 0))(_wait_prev_store)
                    _expand(ib, hj, off, par)
                    pltpu.async_copy(
                        obuf.at[par],
                        out_hbm.at[h, :, pl.ds(b0 + off, SW)],
                        ssem.at[par],
                    )
                return carry2

            lax.fori_loop(0, HG * 2 // NBUF, pair_step, 0)

            @pl.when(g + 2 < ngroups)
            def _prefetch():
                pltpu.async_copy(
                    idx_hbm.at[pl.ds((g + 2) * HG, HG), pl.ds(b0, BW)],
                    idx_v.at[ib], isem.at[ib],
                )
        return carry

    lax.fori_loop(0, ngroups // 2, gp_step, 0)
    for par in range(NBUF):
        pltpu.make_async_copy(
            obuf.at[par], out_hbm.at[0, :, pl.ds(b0, SW)], ssem.at[par]
        ).wait()


def kernel(tc, embedding):
    bsz, hist = tc.shape
    nrows = embedding.shape[0]
    assert bsz % NW == 0 and bsz // NW == BW
    assert hist % (2 * HG) == 0

    idx_t = jnp.transpose(tc).astype(jnp.int32)
    mesh = plsc.VectorSubcoreMesh(
        core_axis_name="c", subcore_axis_name="s", num_cores=NC, num_subcores=NS
    )
    run = pl.kernel(
        functools.partial(_tile_body, hist, nrows),
        out_type=jax.ShapeDtypeStruct((hist, D, bsz), jnp.float32),
        mesh=mesh,
        scratch_types=[
            pltpu.VMEM((2, HG, BW), jnp.int32),
            pltpu.VMEM(embedding.shape, jnp.float32),
            pltpu.VMEM((nrows * TSK,), jnp.float32),
            pltpu.VMEM((NBUF, D, SW), jnp.float32),
            pltpu.SemaphoreType.DMA((NBUF,)),
            pltpu.SemaphoreType.DMA((2,)),
        ],
        compiler_params=pltpu.CompilerParams(needs_layout_passes=False),
    )
    out_t = run(idx_t, embedding)
    return jnp.transpose(out_t, (2, 0, 1))
